# baseline (speedup 1.0000x reference)
import functools

import jax
import jax.numpy as jnp
from jax import lax
from jax.experimental import pallas as pl
from jax.experimental.pallas import tpu as pltpu
from jax.experimental.pallas import tpu_sc as plsc

S = 512
SUB = 1024
GW = 272
CK = 128


def _ln(x, g, b, eps=1e-5):
    n = x.shape[-1]
    s1 = jnp.sum(x, axis=-1, keepdims=True)
    s2 = jnp.sum(x * x, axis=-1, keepdims=True)
    mu = s1 * (1.0 / n)
    var = s2 * (1.0 / n) - mu * mu
    inv = lax.rsqrt(var + eps)
    return (x - mu) * inv * g + b


def _mm(a, b):
    return lax.dot_general(
        a.astype(jnp.bfloat16), b, (((1,), (0,)), ((), ())),
        preferred_element_type=jnp.float32)


def _stage_a(pf_ref, vf_ref, mask_ref,
             pp_W, pp_b, pp_g, pp_be, vp_W, vp_b, vp_g, vp_be,
             att_W1, att_b1, att_g, att_be, att_W2, att_b2,
             geo_W1, geo_b1, geo_g, geo_be, geo_W2, geo_b2,
             pp_out, vp_out, attn_out, gaug_out):
    npts = pf_ref.shape[0]
    for i in range(npts // SUB):
        sl = pl.ds(i * SUB, SUB)
        x = pf_ref[sl, :]
        pp = jax.nn.relu(_ln(_mm(x, pp_W[...]) + pp_b[...], pp_g[...], pp_be[...]))
        v = vf_ref[sl, :]
        vp = jax.nn.relu(_ln(_mm(v, vp_W[...]) + vp_b[...], vp_g[...], vp_be[...]))
        vp = vp * mask_ref[sl, :]
        cat = jnp.concatenate([pp, vp], axis=1)
        h = jax.nn.relu(_ln(_mm(cat, att_W1[...]) + att_b1[...], att_g[...], att_be[...]))
        attn = jax.nn.sigmoid(_mm(h, att_W2[...]) + att_b2[...])
        g1 = jax.nn.relu(_ln(_mm(cat, geo_W1[...]) + geo_b1[...], geo_g[...], geo_be[...]))
        geo = _mm(g1, geo_W2[...]) + geo_b2[...]

        pp_out[sl, :] = pp.astype(jnp.bfloat16)
        vp_out[sl, :] = vp.astype(jnp.bfloat16)
        attn_out[sl, :] = attn
        gg2 = jnp.sum(geo * geo, axis=1, keepdims=True)
        gaug_out[sl, :] = jnp.concatenate(
            [geo, attn, jnp.ones_like(attn), gg2,
             jnp.zeros((SUB, GW - 259), jnp.float32)], axis=1)


def _splat(v, dtype=jnp.int32):
    return jnp.full((16,), v, dtype)


def _sc_segment(gaug_hbm, ids_hbm, zeros_hbm,
                num_hbm, mg2_hbm, geo2_hbm, sattn_hbm, cnt_hbm,
                table, ntable, geo_v, gath_v, gathn_v, ntab_v, idx_v,
                num_v, mg2_v, geo2_v, sattn_v, cnt_v):
    c = lax.axis_index("c")
    s = lax.axis_index("s")
    base = (c * 16 + s) * 1024
    lanes = lax.iota(jnp.int32, 16)

    pltpu.sync_copy(zeros_hbm, table.at[pl.ds(s * CK, CK)])
    plsc.subcore_barrier()

    for k in range(1024 // CK):
        pt = base + k * CK
        pltpu.sync_copy(ids_hbm.at[pl.ds(pt, CK)], idx_v)
        pltpu.sync_copy(gaug_hbm.at[pl.ds(pt, CK)], geo_v)
        pltpu.sync_copy(geo_v, table.at[idx_v], add=True)
    plsc.subcore_barrier()

    pltpu.sync_copy(table.at[pl.ds(s * CK, CK)], geo_v)

    def nbody(gi, _):
        rows = lanes + gi * 16
        acc = jnp.zeros((16,), jnp.float32)
        for k in range(256):
            m = plsc.load_gather(geo_v, [rows, _splat(k)])
            acc = acc + m * m
        plsc.store_scatter(ntab_v, [rows, _splat(0)], acc)
        return 0

    lax.fori_loop(0, CK // 16, nbody, 0)
    pltpu.sync_copy(ntab_v, ntable.at[pl.ds(s * CK, CK)])
    plsc.subcore_barrier()

    for k in range(1024 // CK):
        pt = base + k * CK
        pltpu.sync_copy(ids_hbm.at[pl.ds(pt, CK)], idx_v)
        pltpu.sync_copy(gaug_hbm.at[pl.ds(pt, CK)], geo_v)
        pltpu.sync_copy(table.at[idx_v], gath_v)
        pltpu.sync_copy(ntable.at[idx_v], gathn_v)

        def body(gi, _):
            rows = lanes + gi * 16
            num = jnp.zeros((16,), jnp.float32)
            for q in range(256):
                col = _splat(q)
                g = plsc.load_gather(geo_v, [rows, col])
                m = plsc.load_gather(gath_v, [rows, col])
                num = num + g * m
            sl16 = pl.ds(gi * 16, 16)
            num_v[sl16] = num
            mg2_v[sl16] = plsc.load_gather(gathn_v, [rows, _splat(0)])
            geo2_v[sl16] = plsc.load_gather(geo_v, [rows, _splat(258)])
            sattn_v[sl16] = plsc.load_gather(gath_v, [rows, _splat(256)])
            cnt_v[sl16] = plsc.load_gather(gath_v, [rows, _splat(257)])
            return 0

        lax.fori_loop(0, CK // 16, body, 0)
        pltpu.sync_copy(num_v, num_hbm.at[pl.ds(pt, CK)])
        pltpu.sync_copy(mg2_v, mg2_hbm.at[pl.ds(pt, CK)])
        pltpu.sync_copy(geo2_v, geo2_hbm.at[pl.ds(pt, CK)])
        pltpu.sync_copy(sattn_v, sattn_hbm.at[pl.ds(pt, CK)])
        pltpu.sync_copy(cnt_v, cnt_hbm.at[pl.ds(pt, CK)])


def _stage_b(num_ref, mg2_ref, geo2_ref, sattn_ref, cnt_ref,
             pp_ref, vp_ref, attn_ref,
             fus_W, fus_b, fus_g, fus_be, out_ref):
    num = num_ref[...]
    mg2 = mg2_ref[...]
    geo2 = geo2_ref[...]
    sattn = sattn_ref[...]
    cnt = cnt_ref[...]
    ma = sattn / jnp.maximum(cnt, 1.0)
    dn = jnp.maximum(jnp.sqrt(geo2 * mg2), 1e-8)
    sim = num / dn
    refined = ma + 0.2 * (attn_ref[...] - ma) * sim
    attended = vp_ref[...].astype(jnp.float32) * refined
    comb = jnp.concatenate([pp_ref[...].astype(jnp.float32), attended], axis=1)
    o = _mm(comb, fus_W[...]) + fus_b[...]
    out_ref[...] = _ln(o, fus_g[...], fus_be[...])


@jax.jit
def kernel(point_features, view_features, superpoint_ids, valid_mask,
           pp_W, pp_b, pp_g, pp_be, vp_W, vp_b, vp_g, vp_be,
           att_W1, att_b1, att_g, att_be, att_W2, att_b2,
           geo_W1, geo_b1, geo_g, geo_be, geo_W2, geo_b2,
           fus_W, fus_b, fus_g, fus_be):
    B, NP, PD = point_features.shape
    VD = view_features.shape[-1]
    H = pp_W.shape[1]
    FD = fus_W.shape[1]
    Bn = B * NP

    pf = point_features.reshape(Bn, PD)
    vf = view_features.reshape(Bn, VD)
    mask = valid_mask.astype(jnp.float32).reshape(Bn, 1)
    offids = (superpoint_ids.astype(jnp.int32)
              + (jnp.arange(B, dtype=jnp.int32)[:, None] % 4) * S).reshape(Bn)

    def bf(w):
        return w.astype(jnp.bfloat16)

    def r2(x):
        return x.reshape(1, -1)

    row_spec = lambda w: pl.BlockSpec((NP, w), lambda b: (b, 0))
    full_spec = lambda shp: pl.BlockSpec(shp, lambda b: tuple(0 for _ in shp))

    wspecs = [full_spec(s) for s in
              [(PD, H), (1, H), (1, H), (1, H),
               (VD, H), (1, H), (1, H), (1, H),
               (2 * H, H), (1, H), (1, H), (1, H), (H, 1), (1, 1),
               (2 * H, H), (1, H), (1, H), (1, H), (H, H), (1, H)]]

    pp_o, vp_o, attn_o, gaug = pl.pallas_call(
        _stage_a,
        grid=(B,),
        in_specs=[row_spec(PD), row_spec(VD), row_spec(1)] + wspecs,
        out_specs=[row_spec(H), row_spec(H), row_spec(1), row_spec(GW)],
        out_shape=[
            jax.ShapeDtypeStruct((Bn, H), jnp.bfloat16),
            jax.ShapeDtypeStruct((Bn, H), jnp.bfloat16),
            jax.ShapeDtypeStruct((Bn, 1), jnp.float32),
            jax.ShapeDtypeStruct((Bn, GW), jnp.float32),
        ],
    )(pf, vf, mask,
      bf(pp_W), r2(pp_b), r2(pp_g), r2(pp_be),
      bf(vp_W), r2(vp_b), r2(vp_g), r2(vp_be),
      bf(att_W1), r2(att_b1), r2(att_g), r2(att_be), bf(att_W2), r2(att_b2),
      bf(geo_W1), r2(geo_b1), r2(geo_g), r2(geo_be), bf(geo_W2), r2(geo_b2))

    f1 = jax.ShapeDtypeStruct((Bn,), jnp.float32)
    sc_seg = pl.kernel(
        _sc_segment,
        out_type=(f1, f1, f1, f1, f1),
        mesh=plsc.VectorSubcoreMesh(core_axis_name="c", subcore_axis_name="s"),
        compiler_params=pltpu.CompilerParams(
            use_tc_tiling_on_sc=False, needs_layout_passes=False),
        scratch_types=[
            pltpu.VMEM_SHARED((4 * S, GW), jnp.float32),
            pltpu.VMEM_SHARED((4 * S, 16), jnp.float32),
            pltpu.VMEM((CK, GW), jnp.float32),
            pltpu.VMEM((CK, GW), jnp.float32),
            pltpu.VMEM((CK, 16), jnp.float32),
            pltpu.VMEM((CK, 16), jnp.float32),
            pltpu.VMEM((CK,), jnp.int32),
            pltpu.VMEM((CK,), jnp.float32),
            pltpu.VMEM((CK,), jnp.float32),
            pltpu.VMEM((CK,), jnp.float32),
            pltpu.VMEM((CK,), jnp.float32),
            pltpu.VMEM((CK,), jnp.float32),
        ],
    )
    num, mg2, geo2, sattn, cnt = sc_seg(
        gaug, offids, jnp.zeros((CK, GW), jnp.float32))

    CH = 2048
    rs2 = lambda w: pl.BlockSpec((CH, w), lambda i: (i, 0))
    out = pl.pallas_call(
        _stage_b,
        grid=(Bn // CH,),
        in_specs=[
            rs2(1), rs2(1), rs2(1), rs2(1), rs2(1),
            rs2(H), rs2(H), rs2(1),
            pl.BlockSpec((2 * H, FD), lambda i: (0, 0)),
            pl.BlockSpec((1, FD), lambda i: (0, 0)),
            pl.BlockSpec((1, FD), lambda i: (0, 0)),
            pl.BlockSpec((1, FD), lambda i: (0, 0)),
        ],
        out_specs=rs2(FD),
        out_shape=jax.ShapeDtypeStruct((Bn, FD), jnp.float32),
    )(num.reshape(Bn, 1), mg2.reshape(Bn, 1), geo2.reshape(Bn, 1),
      sattn.reshape(Bn, 1), cnt.reshape(Bn, 1),
      pp_o, vp_o, attn_o,
      bf(fus_W), r2(fus_b), r2(fus_g), r2(fus_be))

    return out.reshape(B, NP, FD)


# speedup vs baseline: 1.2908x; 1.2908x over previous
import functools

import jax
import jax.numpy as jnp
from jax import lax
from jax.experimental import pallas as pl
from jax.experimental.pallas import tpu as pltpu
from jax.experimental.pallas import tpu_sc as plsc

S = 512
SUB = 1024
GW = 272
CK = 128


def _ln(x, g, b, eps=1e-5):
    n = x.shape[-1]
    s1 = jnp.sum(x, axis=-1, keepdims=True)
    s2 = jnp.sum(x * x, axis=-1, keepdims=True)
    mu = s1 * (1.0 / n)
    var = s2 * (1.0 / n) - mu * mu
    inv = lax.rsqrt(var + eps)
    return (x - mu) * inv * g + b


def _mm(a, b):
    return lax.dot_general(
        a.astype(jnp.bfloat16), b, (((1,), (0,)), ((), ())),
        preferred_element_type=jnp.float32)


def _stage_a(pf_ref, vf_ref, mask_ref,
             pp_W, pp_b, pp_g, pp_be, vp_W, vp_b, vp_g, vp_be,
             att_W1, att_b1, att_g, att_be, att_W2, att_b2,
             geo_W1, geo_b1, geo_g, geo_be, geo_W2, geo_b2,
             pp_out, vp_out, gaug_out):
    npts = pf_ref.shape[0]
    for i in range(npts // SUB):
        sl = pl.ds(i * SUB, SUB)
        x = pf_ref[sl, :]
        pp = jax.nn.relu(_ln(_mm(x, pp_W[...]) + pp_b[...], pp_g[...], pp_be[...]))
        v = vf_ref[sl, :]
        vp = jax.nn.relu(_ln(_mm(v, vp_W[...]) + vp_b[...], vp_g[...], vp_be[...]))
        vp = vp * mask_ref[sl, :]
        cat = jnp.concatenate([pp, vp], axis=1)
        h = jax.nn.relu(_ln(_mm(cat, att_W1[...]) + att_b1[...], att_g[...], att_be[...]))
        attn = jax.nn.sigmoid(_mm(h, att_W2[...]) + att_b2[...])
        g1 = jax.nn.relu(_ln(_mm(cat, geo_W1[...]) + geo_b1[...], geo_g[...], geo_be[...]))
        geo = _mm(g1, geo_W2[...]) + geo_b2[...]

        pp_out[sl, :] = pp.astype(jnp.bfloat16)
        vp_out[sl, :] = vp.astype(jnp.bfloat16)
        gg2 = jnp.sum(geo * geo, axis=1, keepdims=True)
        gaug_out[sl, :] = jnp.concatenate(
            [geo, attn, jnp.ones_like(attn), gg2,
             jnp.zeros((SUB, GW - 259), jnp.float32)], axis=1)


def _sc_segment(gaug_hbm, ids_hbm, zeros_hbm, mg_hbm,
                table, geo_v, gath_v, idx_v):
    c = lax.axis_index("c")
    s = lax.axis_index("s")
    base = (c * 16 + s) * 1024

    pltpu.sync_copy(zeros_hbm, table.at[pl.ds(s * CK, CK)])
    plsc.subcore_barrier()

    for k in range(1024 // CK):
        pt = base + k * CK
        pltpu.sync_copy(ids_hbm.at[pl.ds(pt, CK)], idx_v)
        pltpu.sync_copy(gaug_hbm.at[pl.ds(pt, CK)], geo_v)
        pltpu.sync_copy(geo_v, table.at[idx_v], add=True)
    plsc.subcore_barrier()

    for k in range(1024 // CK):
        pt = base + k * CK
        pltpu.sync_copy(ids_hbm.at[pl.ds(pt, CK)], idx_v)
        pltpu.sync_copy(table.at[idx_v], gath_v)
        pltpu.sync_copy(gath_v, mg_hbm.at[pl.ds(pt, CK)])


def _stage_b(gaug_ref, mgrow_ref, pp_ref, vp_ref,
             fus_W, fus_b, fus_g, fus_be, out_ref):
    gaug = gaug_ref[...]
    mgrow = mgrow_ref[...]
    geo = gaug[:, 0:256]
    mg = mgrow[:, 0:256]
    attn = gaug[:, 256:257]
    geo2 = gaug[:, 258:259]
    sattn = mgrow[:, 256:257]
    cnt = mgrow[:, 257:258]
    ma = sattn / jnp.maximum(cnt, 1.0)
    num = jnp.sum(geo * mg, axis=1, keepdims=True)
    mg2 = jnp.sum(mg * mg, axis=1, keepdims=True)
    dn = jnp.maximum(jnp.sqrt(geo2 * mg2), 1e-8)
    sim = num / dn
    refined = ma + 0.2 * (attn - ma) * sim
    attended = vp_ref[...].astype(jnp.float32) * refined
    comb = jnp.concatenate([pp_ref[...].astype(jnp.float32), attended], axis=1)
    o = _mm(comb, fus_W[...]) + fus_b[...]
    out_ref[...] = _ln(o, fus_g[...], fus_be[...])


@jax.jit
def kernel(point_features, view_features, superpoint_ids, valid_mask,
           pp_W, pp_b, pp_g, pp_be, vp_W, vp_b, vp_g, vp_be,
           att_W1, att_b1, att_g, att_be, att_W2, att_b2,
           geo_W1, geo_b1, geo_g, geo_be, geo_W2, geo_b2,
           fus_W, fus_b, fus_g, fus_be):
    B, NP, PD = point_features.shape
    VD = view_features.shape[-1]
    H = pp_W.shape[1]
    FD = fus_W.shape[1]
    Bn = B * NP

    pf = point_features.reshape(Bn, PD)
    vf = view_features.reshape(Bn, VD)
    mask = valid_mask.astype(jnp.float32).reshape(Bn, 1)
    offids = (superpoint_ids.astype(jnp.int32)
              + (jnp.arange(B, dtype=jnp.int32)[:, None] % 4) * S).reshape(Bn)

    def bf(w):
        return w.astype(jnp.bfloat16)

    def r2(x):
        return x.reshape(1, -1)

    row_spec = lambda w: pl.BlockSpec((NP, w), lambda b: (b, 0))
    full_spec = lambda shp: pl.BlockSpec(shp, lambda b: tuple(0 for _ in shp))

    wspecs = [full_spec(s) for s in
              [(PD, H), (1, H), (1, H), (1, H),
               (VD, H), (1, H), (1, H), (1, H),
               (2 * H, H), (1, H), (1, H), (1, H), (H, 1), (1, 1),
               (2 * H, H), (1, H), (1, H), (1, H), (H, H), (1, H)]]

    pp_o, vp_o, gaug = pl.pallas_call(
        _stage_a,
        grid=(B,),
        in_specs=[row_spec(PD), row_spec(VD), row_spec(1)] + wspecs,
        out_specs=[row_spec(H), row_spec(H), row_spec(GW)],
        out_shape=[
            jax.ShapeDtypeStruct((Bn, H), jnp.bfloat16),
            jax.ShapeDtypeStruct((Bn, H), jnp.bfloat16),
            jax.ShapeDtypeStruct((Bn, GW), jnp.float32),
        ],
    )(pf, vf, mask,
      bf(pp_W), r2(pp_b), r2(pp_g), r2(pp_be),
      bf(vp_W), r2(vp_b), r2(vp_g), r2(vp_be),
      bf(att_W1), r2(att_b1), r2(att_g), r2(att_be), bf(att_W2), r2(att_b2),
      bf(geo_W1), r2(geo_b1), r2(geo_g), r2(geo_be), bf(geo_W2), r2(geo_b2))

    sc_seg = pl.kernel(
        _sc_segment,
        out_type=jax.ShapeDtypeStruct((Bn, GW), jnp.float32),
        mesh=plsc.VectorSubcoreMesh(core_axis_name="c", subcore_axis_name="s"),
        compiler_params=pltpu.CompilerParams(
            use_tc_tiling_on_sc=False, needs_layout_passes=False),
        scratch_types=[
            pltpu.VMEM_SHARED((4 * S, GW), jnp.float32),
            pltpu.VMEM((CK, GW), jnp.float32),
            pltpu.VMEM((CK, GW), jnp.float32),
            pltpu.VMEM((CK,), jnp.int32),
        ],
    )
    mgrow = sc_seg(gaug, offids, jnp.zeros((CK, GW), jnp.float32))

    CH = 2048
    rs2 = lambda w: pl.BlockSpec((CH, w), lambda i: (i, 0))
    out = pl.pallas_call(
        _stage_b,
        grid=(Bn // CH,),
        in_specs=[
            rs2(GW), rs2(GW), rs2(H), rs2(H),
            pl.BlockSpec((2 * H, FD), lambda i: (0, 0)),
            pl.BlockSpec((1, FD), lambda i: (0, 0)),
            pl.BlockSpec((1, FD), lambda i: (0, 0)),
            pl.BlockSpec((1, FD), lambda i: (0, 0)),
        ],
        out_specs=rs2(FD),
        out_shape=jax.ShapeDtypeStruct((Bn, FD), jnp.float32),
    )(gaug, mgrow, pp_o, vp_o,
      bf(fus_W), r2(fus_b), r2(fus_g), r2(fus_be))

    return out.reshape(B, NP, FD)


# speedup vs baseline: 1.9281x; 1.4937x over previous
import jax
import jax.numpy as jnp
from jax.experimental import pallas as pl
from jax.experimental.pallas import tpu as pltpu

S = 512
SUB = 1024


def _ln(x, g, b, eps=1e-5):
    n = x.shape[-1]
    s1 = jnp.sum(x, axis=-1, keepdims=True)
    s2 = jnp.sum(x * x, axis=-1, keepdims=True)
    mu = s1 * (1.0 / n)
    var = s2 * (1.0 / n) - mu * mu
    inv = jax.lax.rsqrt(var + eps)
    return (x - mu) * inv * g + b


def _mm(a, b):
    return jax.lax.dot_general(
        a.astype(jnp.bfloat16), b, (((1,), (0,)), ((), ())),
        preferred_element_type=jnp.float32)


def _fused(pf_ref, vf_ref, ids_ref, mask_ref,
           pp_W, pp_b, pp_g, pp_be, vp_W, vp_b, vp_g, vp_be,
           att_W1, att_b1, att_g, att_be, att_W2, att_b2,
           geo_W1, geo_b1, geo_g, geo_be, geo_W2, geo_b2,
           fus_W, fus_b, fus_g, fus_be,
           out_ref,
           pp_s, vp_s, geo_s, attn_s, oh_s, sgeo_s, saux_s):
    npts = pf_ref.shape[0]
    nsub = npts // SUB

    sgeo_acc = None
    saux_acc = None
    for i in range(nsub):
        sl = pl.ds(i * SUB, SUB)
        x = pf_ref[sl, :]
        pp = jax.nn.relu(_ln(_mm(x, pp_W[...]) + pp_b[...], pp_g[...], pp_be[...]))
        v = vf_ref[sl, :]
        vp = jax.nn.relu(_ln(_mm(v, vp_W[...]) + vp_b[...], vp_g[...], vp_be[...]))
        vp = vp * mask_ref[sl, :]
        cat = jnp.concatenate([pp, vp], axis=1)
        h = jax.nn.relu(_ln(_mm(cat, att_W1[...]) + att_b1[...], att_g[...], att_be[...]))
        attn = jax.nn.sigmoid(_mm(h, att_W2[...]) + att_b2[...])
        g1 = jax.nn.relu(_ln(_mm(cat, geo_W1[...]) + geo_b1[...], geo_g[...], geo_be[...]))
        geo = _mm(g1, geo_W2[...]) + geo_b2[...]

        pp_s[sl, :] = pp.astype(jnp.bfloat16)
        vp_s[sl, :] = vp.astype(jnp.bfloat16)
        geo_s[sl, :] = geo
        attn_s[sl, :] = attn

        ids = ids_ref[sl, :]
        iota = jax.lax.broadcasted_iota(jnp.int32, (SUB, S), 1)
        oh_bf = (ids == iota).astype(jnp.bfloat16)
        oh_s[sl, :] = oh_bf
        psgeo = jax.lax.dot_general(
            oh_bf, geo.astype(jnp.bfloat16),
            (((0,), (0,)), ((), ())), preferred_element_type=jnp.float32)
        aux = jnp.concatenate([attn, jnp.ones_like(attn)], axis=1)
        psaux = jax.lax.dot_general(
            oh_bf, aux.astype(jnp.bfloat16), (((0,), (0,)), ((), ())),
            preferred_element_type=jnp.float32)
        sgeo_acc = psgeo if i == 0 else sgeo_acc + psgeo
        saux_acc = psaux if i == 0 else saux_acc + psaux

    sgeo_s[...] = sgeo_acc.astype(jnp.bfloat16)
    saux_s[...] = saux_acc.astype(jnp.bfloat16)

    for i in range(nsub):
        sl = pl.ds(i * SUB, SUB)
        oh_bf = oh_s[sl, :]
        mg = jax.lax.dot_general(
            oh_bf, sgeo_s[...], (((1,), (0,)), ((), ())),
            preferred_element_type=jnp.float32)
        aux = jax.lax.dot_general(
            oh_bf, saux_s[...], (((1,), (0,)), ((), ())),
            preferred_element_type=jnp.float32)
        ma = aux[:, 0:1] / jnp.maximum(aux[:, 1:2], 1.0)

        geo = geo_s[sl, :]
        num = jnp.sum(geo * mg, axis=1, keepdims=True)
        dn = jnp.sqrt(jnp.sum(geo * geo, axis=1, keepdims=True)) * \
             jnp.sqrt(jnp.sum(mg * mg, axis=1, keepdims=True))
        dn = jnp.maximum(dn, 1e-8)
        sim = num / dn
        refined = ma + 0.2 * (attn_s[sl, :] - ma) * sim

        attended = vp_s[sl, :].astype(jnp.float32) * refined
        comb = jnp.concatenate([pp_s[sl, :].astype(jnp.float32), attended], axis=1)
        o = _mm(comb, fus_W[...]) + fus_b[...]
        out_ref[sl, :] = _ln(o, fus_g[...], fus_be[...])


@jax.jit
def kernel(point_features, view_features, superpoint_ids, valid_mask,
           pp_W, pp_b, pp_g, pp_be, vp_W, vp_b, vp_g, vp_be,
           att_W1, att_b1, att_g, att_be, att_W2, att_b2,
           geo_W1, geo_b1, geo_g, geo_be, geo_W2, geo_b2,
           fus_W, fus_b, fus_g, fus_be):
    B, NP, PD = point_features.shape
    VD = view_features.shape[-1]
    H = pp_W.shape[1]
    FD = fus_W.shape[1]
    Bn = B * NP

    pf = point_features.reshape(Bn, PD)
    vf = view_features.reshape(Bn, VD)
    ids = superpoint_ids.astype(jnp.int32).reshape(Bn, 1)
    mask = valid_mask.astype(jnp.float32).reshape(Bn, 1)

    def bf(w):
        return w.astype(jnp.bfloat16)

    def r2(x):
        return x.reshape(1, -1)

    row_spec = lambda w: pl.BlockSpec((NP, w), lambda b: (b, 0))
    full_spec = lambda shp: pl.BlockSpec(shp, lambda b: tuple(0 for _ in shp))

    wspecs = [full_spec(s) for s in
              [(PD, H), (1, H), (1, H), (1, H),
               (VD, H), (1, H), (1, H), (1, H),
               (2 * H, H), (1, H), (1, H), (1, H), (H, 1), (1, 1),
               (2 * H, H), (1, H), (1, H), (1, H), (H, H), (1, H),
               (2 * H, FD), (1, FD), (1, FD), (1, FD)]]

    out = pl.pallas_call(
        _fused,
        grid=(B,),
        in_specs=[row_spec(PD), row_spec(VD), row_spec(1), row_spec(1)] + wspecs,
        out_specs=row_spec(FD),
        out_shape=jax.ShapeDtypeStruct((Bn, FD), jnp.float32),
        scratch_shapes=[
            pltpu.VMEM((NP, H), jnp.bfloat16),
            pltpu.VMEM((NP, H), jnp.bfloat16),
            pltpu.VMEM((NP, H), jnp.float32),
            pltpu.VMEM((NP, 1), jnp.float32),
            pltpu.VMEM((NP, S), jnp.bfloat16),
            pltpu.VMEM((S, H), jnp.bfloat16),
            pltpu.VMEM((S, 2), jnp.bfloat16),
        ],
    )(pf, vf, ids, mask,
      bf(pp_W), r2(pp_b), r2(pp_g), r2(pp_be),
      bf(vp_W), r2(vp_b), r2(vp_g), r2(vp_be),
      bf(att_W1), r2(att_b1), r2(att_g), r2(att_be), bf(att_W2), r2(att_b2),
      bf(geo_W1), r2(geo_b1), r2(geo_g), r2(geo_be), bf(geo_W2), r2(geo_b2),
      bf(fus_W), r2(fus_b), r2(fus_g), r2(fus_be))

    return out.reshape(B, NP, FD)


# speedup vs baseline: 2.3566x; 1.2222x over previous
import jax
import jax.numpy as jnp
from jax.experimental import pallas as pl
from jax.experimental.pallas import tpu as pltpu

S = 512
SUB = 1024


def _ln(x, g, b, eps=1e-5):
    n = x.shape[-1]
    s1 = jnp.sum(x, axis=-1, keepdims=True)
    s2 = jnp.sum(x * x, axis=-1, keepdims=True)
    mu = s1 * (1.0 / n)
    var = s2 * (1.0 / n) - mu * mu
    inv = jax.lax.rsqrt(var + eps)
    return (x - mu) * inv * g + b


def _mm(a, b):
    return jax.lax.dot_general(
        a.astype(jnp.bfloat16), b, (((1,), (0,)), ((), ())),
        preferred_element_type=jnp.float32)


def _fused(pf_ref, vf_ref, ids_ref, mask_ref,
           pp_W, pp_b, pp_g, pp_be, vp_W, vp_b, vp_g, vp_be,
           att_W1, att_b1, att_g, att_be, att_W2, att_b2,
           geo_W1, geo_b1, geo_g, geo_be, geo_W2, geo_b2,
           fus_W, fus_b, fus_g, fus_be,
           out_ref,
           pp_s, vp_s, geo_s, attn_s, sgeo_s, saux_s):
    npts = pf_ref.shape[0]
    nsub = npts // SUB

    sgeo_acc = None
    saux_acc = None
    for i in range(nsub):
        sl = pl.ds(i * SUB, SUB)
        x = pf_ref[sl, :]
        pp = jax.nn.relu(_ln(_mm(x, pp_W[...]) + pp_b[...], pp_g[...], pp_be[...]))
        v = vf_ref[sl, :]
        vp = jax.nn.relu(_ln(_mm(v, vp_W[...]) + vp_b[...], vp_g[...], vp_be[...]))
        vp = vp * mask_ref[sl, :]
        cat = jnp.concatenate([pp, vp], axis=1)
        h = jax.nn.relu(_ln(_mm(cat, att_W1[...]) + att_b1[...], att_g[...], att_be[...]))
        attn = jax.nn.sigmoid(_mm(h, att_W2[...]) + att_b2[...])
        g1 = jax.nn.relu(_ln(_mm(cat, geo_W1[...]) + geo_b1[...], geo_g[...], geo_be[...]))
        geo = _mm(g1, geo_W2[...]) + geo_b2[...]

        pp_s[sl, :] = pp.astype(jnp.bfloat16)
        vp_s[sl, :] = vp.astype(jnp.bfloat16)
        geo_s[sl, :] = geo
        attn_s[sl, :] = attn

        ids = ids_ref[sl, :]
        iota = jax.lax.broadcasted_iota(jnp.int32, (SUB, S), 1)
        oh_bf = (ids == iota).astype(jnp.bfloat16)
        psgeo = jax.lax.dot_general(
            oh_bf, geo.astype(jnp.bfloat16),
            (((0,), (0,)), ((), ())), preferred_element_type=jnp.float32)
        aux = jnp.concatenate([attn, jnp.ones_like(attn)], axis=1)
        psaux = jax.lax.dot_general(
            oh_bf, aux.astype(jnp.bfloat16), (((0,), (0,)), ((), ())),
            preferred_element_type=jnp.float32)
        sgeo_acc = psgeo if i == 0 else sgeo_acc + psgeo
        saux_acc = psaux if i == 0 else saux_acc + psaux

    sgeo_s[...] = sgeo_acc.astype(jnp.bfloat16)
    saux_s[...] = saux_acc.astype(jnp.bfloat16)

    for i in range(nsub):
        sl = pl.ds(i * SUB, SUB)
        ids = ids_ref[sl, :]
        iota = jax.lax.broadcasted_iota(jnp.int32, (SUB, S), 1)
        oh_bf = (ids == iota).astype(jnp.bfloat16)
        mg = jax.lax.dot_general(
            oh_bf, sgeo_s[...], (((1,), (0,)), ((), ())),
            preferred_element_type=jnp.float32)
        aux = jax.lax.dot_general(
            oh_bf, saux_s[...], (((1,), (0,)), ((), ())),
            preferred_element_type=jnp.float32)
        ma = aux[:, 0:1] / jnp.maximum(aux[:, 1:2], 1.0)

        geo = geo_s[sl, :]
        num = jnp.sum(geo * mg, axis=1, keepdims=True)
        dn = jnp.sqrt(jnp.sum(geo * geo, axis=1, keepdims=True)) * \
             jnp.sqrt(jnp.sum(mg * mg, axis=1, keepdims=True))
        dn = jnp.maximum(dn, 1e-8)
        sim = num / dn
        refined = ma + 0.2 * (attn_s[sl, :] - ma) * sim

        attended = vp_s[sl, :].astype(jnp.float32) * refined
        comb = jnp.concatenate([pp_s[sl, :].astype(jnp.float32), attended], axis=1)
        o = _mm(comb, fus_W[...]) + fus_b[...]
        out_ref[sl, :] = _ln(o, fus_g[...], fus_be[...])


@jax.jit
def kernel(point_features, view_features, superpoint_ids, valid_mask,
           pp_W, pp_b, pp_g, pp_be, vp_W, vp_b, vp_g, vp_be,
           att_W1, att_b1, att_g, att_be, att_W2, att_b2,
           geo_W1, geo_b1, geo_g, geo_be, geo_W2, geo_b2,
           fus_W, fus_b, fus_g, fus_be):
    B, NP, PD = point_features.shape
    VD = view_features.shape[-1]
    H = pp_W.shape[1]
    FD = fus_W.shape[1]
    Bn = B * NP

    pf = point_features.reshape(Bn, PD)
    vf = view_features.reshape(Bn, VD)
    ids = superpoint_ids.astype(jnp.int32).reshape(Bn, 1)
    mask = valid_mask.astype(jnp.float32).reshape(Bn, 1)

    def bf(w):
        return w.astype(jnp.bfloat16)

    def r2(x):
        return x.reshape(1, -1)

    row_spec = lambda w: pl.BlockSpec((NP, w), lambda b: (b, 0))
    full_spec = lambda shp: pl.BlockSpec(shp, lambda b: tuple(0 for _ in shp))

    wspecs = [full_spec(s) for s in
              [(PD, H), (1, H), (1, H), (1, H),
               (VD, H), (1, H), (1, H), (1, H),
               (2 * H, H), (1, H), (1, H), (1, H), (H, 1), (1, 1),
               (2 * H, H), (1, H), (1, H), (1, H), (H, H), (1, H),
               (2 * H, FD), (1, FD), (1, FD), (1, FD)]]

    out = pl.pallas_call(
        _fused,
        grid=(B,),
        in_specs=[row_spec(PD), row_spec(VD), row_spec(1), row_spec(1)] + wspecs,
        out_specs=row_spec(FD),
        out_shape=jax.ShapeDtypeStruct((Bn, FD), jnp.float32),
        scratch_shapes=[
            pltpu.VMEM((NP, H), jnp.bfloat16),
            pltpu.VMEM((NP, H), jnp.bfloat16),
            pltpu.VMEM((NP, H), jnp.float32),
            pltpu.VMEM((NP, 1), jnp.float32),
            pltpu.VMEM((S, H), jnp.bfloat16),
            pltpu.VMEM((S, 2), jnp.bfloat16),
        ],
    )(pf, vf, ids, mask,
      bf(pp_W), r2(pp_b), r2(pp_g), r2(pp_be),
      bf(vp_W), r2(vp_b), r2(vp_g), r2(vp_be),
      bf(att_W1), r2(att_b1), r2(att_g), r2(att_be), bf(att_W2), r2(att_b2),
      bf(geo_W1), r2(geo_b1), r2(geo_g), r2(geo_be), bf(geo_W2), r2(geo_b2),
      bf(fus_W), r2(fus_b), r2(fus_g), r2(fus_be))

    return out.reshape(B, NP, FD)
